# baseline (speedup 1.0000x reference)
import jax
import jax.numpy as jnp
from jax.experimental import pallas as pl
from jax.experimental.pallas import tpu as pltpu

_R, _C = 160, 128
_P = _R * _C
_G = _R // 8
_MOUT = 200
_BIG = 2**30
_B = 8


def _tile_iota():
    return (jax.lax.broadcasted_iota(jnp.int32, (8, _C), 0) * _C
            + jax.lax.broadcasted_iota(jnp.int32, (8, _C), 1))


def _combine(a, b):
    take_b = (b[0] > a[0]) | ((b[0] == a[0]) & (b[1] < a[1]))
    return (jnp.where(take_b, b[0], a[0]), jnp.where(take_b, b[1], a[1]))


def _lex_chain(t):
    for axis, shifts in ((1, (1, 2, 4, 8, 16, 32, 64)), (0, (1, 2, 4))):
        for sh in shifts:
            r = (pltpu.roll(t[0], sh, axis), pltpu.roll(t[1], sh, axis))
            t = _combine(t, r)
    return t


def _sum_bc(x, ones):
    x = jax.lax.dot(x, ones, precision=jax.lax.Precision.HIGHEST,
                    preferred_element_type=jnp.float32)
    for sh in (1, 2, 4):
        x = x + pltpu.roll(x, sh, 0)
    return x


def _nms_kernel(thr_ref, x1, y1, x2, y2, s, sel_ref, num_ref, ws, ar, ones):
    iou_thr = thr_ref[0, 0]
    score_thr = thr_ref[1, 0]
    ws[...] = jnp.where(s[...] > score_thr, s[...], -jnp.inf)
    ar[...] = (x2[...] - x1[...]) * (y2[...] - y1[...])
    ones[...] = jnp.full((_C, _C), 1.0, jnp.float32)

    ti = _tile_iota()

    def tree_sweep(i_bc, valid_v):
        acc0 = acc1 = None
        coords = None
        zero = jnp.float32(0.0)
        for g in range(_G):
            gs = pl.ds(g * 8, 8)
            w_g = ws[gs, :]
            if i_bc is not None:
                pick_g = (ti + g * 1024) == i_bc
                w_g = jnp.where(valid_v & pick_g, -jnp.inf, w_g)
                ws[gs, :] = w_g
                cs = (jnp.where(pick_g, x1[gs, :], zero),
                      jnp.where(pick_g, y1[gs, :], zero),
                      jnp.where(pick_g, x2[gs, :], zero),
                      jnp.where(pick_g, y2[gs, :], zero),
                      jnp.where(pick_g, ar[gs, :], zero))
                coords = cs if coords is None else tuple(
                    c + d for c, d in zip(coords, cs))
            node = (w_g, ti + g * 1024)
            if g % 2 == 0:
                acc0 = node if acc0 is None else _combine(acc0, node)
            else:
                acc1 = node if acc1 is None else _combine(acc1, node)
        return _combine(acc0, acc1), coords

    top0, _ = tree_sweep(None, None)

    def round_fn(state):
        num_v, stopped_v, sel, sx1, sy1, sx2, sy2, sa, top = state
        m_bc, i_bc = _lex_chain(top)
        valid_v = (m_bc > -jnp.inf) & (stopped_v == 0)
        top, coords = tree_sweep(i_bc, valid_v)
        one = ones[...]
        b0 = _sum_bc(coords[0], one)
        b1 = _sum_bc(coords[1], one)
        b2 = _sum_bc(coords[2], one)
        b3 = _sum_bc(coords[3], one)
        a = _sum_bc(coords[4], one)
        xx1 = jnp.maximum(b0, sx1)
        yy1 = jnp.maximum(b1, sy1)
        xx2 = jnp.minimum(b2, sx2)
        yy2 = jnp.minimum(b3, sy2)
        inter = (jnp.clip(xx2 - xx1, 0.0, None)
                 * jnp.clip(yy2 - yy1, 0.0, None))
        union = jnp.maximum(a + sa - inter, 1e-6)
        iou = inter / union
        hit = (iou >= iou_thr) & (ti < num_v)
        hitcnt = _sum_bc(jnp.where(hit, 1.0, 0.0), one)
        accepted = valid_v & (hitcnt == 0.0) & (num_v < _MOUT)
        slot = accepted & (ti == num_v)
        sel = jnp.where(slot, i_bc, sel)
        sx1 = jnp.where(slot, b0, sx1)
        sy1 = jnp.where(slot, b1, sy1)
        sx2 = jnp.where(slot, b2, sx2)
        sy2 = jnp.where(slot, b3, sy2)
        sa = jnp.where(slot, a, sa)
        num_v = num_v + accepted.astype(jnp.int32)
        stopped_v = jnp.maximum(stopped_v,
                                (m_bc == -jnp.inf).astype(jnp.int32))
        return (num_v, stopped_v, sel, sx1, sy1, sx2, sy2, sa, top)

    def cond(carry):
        num_s, stop_s = carry[0], carry[1]
        return jnp.logical_and(num_s < _MOUT, jnp.logical_not(stop_s))

    def body(carry):
        state = carry[2:10] + (carry[10:],)
        for _ in range(_B):
            state = round_fn(state)
        enc = state[0] + state[1] * 65536
        e = jnp.max(enc)
        return (jnp.bitwise_and(e, 65535), e >= 65536) + state[:8] + state[8]

    zf = jnp.zeros((8, _C), jnp.float32)
    zi = jnp.zeros((8, _C), jnp.int32)
    carry = (jnp.int32(0), jnp.bool_(False),
             zi, zi, zi, zf, zf, zf, zf, zf) + top0
    carry = jax.lax.while_loop(cond, body, carry)
    sel_ref[...] = carry[4]
    num_ref[0, 0] = carry[0]


def kernel(boxes, scores, max_output_size, iou_threshold, scores_threshold):
    boxes = boxes.astype(jnp.float32)
    scores = scores.astype(jnp.float32)
    n = boxes.shape[0]
    pad = _P - n
    bx = jnp.pad(boxes, ((0, pad), (0, 0)))
    planes = bx.T.reshape(4, _R, _C)
    s = jnp.pad(scores, (0, pad), constant_values=-jnp.inf).reshape(_R, _C)
    thr = jnp.stack([jnp.asarray(iou_threshold, jnp.float32),
                     jnp.asarray(scores_threshold, jnp.float32)]).reshape(2, 1)

    sel_m, num_m = pl.pallas_call(
        _nms_kernel,
        in_specs=[
            pl.BlockSpec(memory_space=pltpu.SMEM),
            pl.BlockSpec(memory_space=pltpu.VMEM),
            pl.BlockSpec(memory_space=pltpu.VMEM),
            pl.BlockSpec(memory_space=pltpu.VMEM),
            pl.BlockSpec(memory_space=pltpu.VMEM),
            pl.BlockSpec(memory_space=pltpu.VMEM),
        ],
        out_specs=[
            pl.BlockSpec(memory_space=pltpu.VMEM),
            pl.BlockSpec(memory_space=pltpu.SMEM),
        ],
        out_shape=[
            jax.ShapeDtypeStruct((8, _C), jnp.int32),
            jax.ShapeDtypeStruct((1, 1), jnp.int32),
        ],
        scratch_shapes=[
            pltpu.VMEM((_R, _C), jnp.float32),
            pltpu.VMEM((_R, _C), jnp.float32),
            pltpu.VMEM((_C, _C), jnp.float32),
        ],
    )(thr, planes[0], planes[1], planes[2], planes[3], s)

    sel = sel_m.reshape(-1)[:_MOUT]
    num = jnp.minimum(num_m[0, 0], jnp.asarray(max_output_size, jnp.int32))
    return (sel, num)


# speedup vs baseline: 1.3421x; 1.3421x over previous
import jax
import jax.numpy as jnp
from jax.experimental import pallas as pl
from jax.experimental.pallas import tpu as pltpu

_R, _C = 160, 128
_P = _R * _C
_G = _R // 8
_MOUT = 200
_BIG = 2**30
_B = 8


def _tile_iota():
    return (jax.lax.broadcasted_iota(jnp.int32, (8, _C), 0) * _C
            + jax.lax.broadcasted_iota(jnp.int32, (8, _C), 1))


def _combine(a, b):
    take_b = (b[0] > a[0]) | ((b[0] == a[0]) & (b[1] < a[1]))
    return (jnp.where(take_b, b[0], a[0]), jnp.where(take_b, b[1], a[1]))


def _nms_kernel(thr_ref, x1, y1, x2, y2, s, sel_ref, num_ref, ws, ar):
    iou_thr = thr_ref[0, 0]
    score_thr = thr_ref[1, 0]
    ws[...] = jnp.where(s[...] > score_thr, s[...], -jnp.inf)
    ar[...] = (x2[...] - x1[...]) * (y2[...] - y1[...])

    ti = _tile_iota()

    def tree_sweep(idx, valid):
        acc0 = acc1 = None
        for g in range(_G):
            gs = pl.ds(g * 8, 8)
            w_g = ws[gs, :]
            if idx is not None:
                pick_g = ((ti + g * 1024) == idx) & valid
                w_g = jnp.where(pick_g, -jnp.inf, w_g)
                ws[gs, :] = w_g
            node = (w_g, ti + g * 1024)
            if g % 2 == 0:
                acc0 = node if acc0 is None else _combine(acc0, node)
            else:
                acc1 = node if acc1 is None else _combine(acc1, node)
        return _combine(acc0, acc1)

    m_v0, i_v0 = tree_sweep(None, None)

    def round_fn(state):
        num, stop, sel, sx1, sy1, sx2, sy2, sa, m_v, i_v = state
        m = jnp.max(m_v)
        valid = m > -jnp.inf
        idx = jnp.min(jnp.where(m_v == m, i_v, _BIG))
        gi = idx // 1024
        base = pl.multiple_of(gi * 8, 8)
        pick = ti == (idx - gi * 1024)
        zero = jnp.float32(0.0)
        b0 = jnp.sum(jnp.where(pick, x1[pl.ds(base, 8), :], zero))
        b1 = jnp.sum(jnp.where(pick, y1[pl.ds(base, 8), :], zero))
        b2 = jnp.sum(jnp.where(pick, x2[pl.ds(base, 8), :], zero))
        b3 = jnp.sum(jnp.where(pick, y2[pl.ds(base, 8), :], zero))
        a = jnp.sum(jnp.where(pick, ar[pl.ds(base, 8), :], zero))
        xx1 = jnp.maximum(b0, sx1)
        yy1 = jnp.maximum(b1, sy1)
        xx2 = jnp.minimum(b2, sx2)
        yy2 = jnp.minimum(b3, sy2)
        inter = (jnp.clip(xx2 - xx1, 0.0, None)
                 * jnp.clip(yy2 - yy1, 0.0, None))
        union = jnp.maximum(a + sa - inter, 1e-6)
        iou = inter / union
        hit = (iou >= iou_thr) & (ti < num)
        hitcnt = jnp.sum(jnp.where(hit, 1.0, zero))
        accepted = valid & (hitcnt == zero) & (num < _MOUT)
        slot = accepted & (ti == num)
        sel = jnp.where(slot, idx, sel)
        sx1 = jnp.where(slot, b0, sx1)
        sy1 = jnp.where(slot, b1, sy1)
        sx2 = jnp.where(slot, b2, sx2)
        sy2 = jnp.where(slot, b3, sy2)
        sa = jnp.where(slot, a, sa)
        num = num + accepted.astype(jnp.int32)
        m_v, i_v = tree_sweep(idx, valid)
        stop = jnp.logical_not(valid)
        return (num, stop, sel, sx1, sy1, sx2, sy2, sa, m_v, i_v)

    def cond(carry):
        return jnp.logical_and(carry[0] < _MOUT, jnp.logical_not(carry[1]))

    def body(carry):
        state = carry
        for _ in range(_B):
            state = round_fn(state)
        return state

    zf = jnp.zeros((8, _C), jnp.float32)
    carry = (jnp.int32(0), jnp.bool_(False),
             jnp.zeros((8, _C), jnp.int32), zf, zf, zf, zf, zf, m_v0, i_v0)
    carry = jax.lax.while_loop(cond, body, carry)
    sel_ref[...] = carry[2]
    num_ref[0, 0] = carry[0]


def kernel(boxes, scores, max_output_size, iou_threshold, scores_threshold):
    boxes = boxes.astype(jnp.float32)
    scores = scores.astype(jnp.float32)
    n = boxes.shape[0]
    pad = _P - n
    bx = jnp.pad(boxes, ((0, pad), (0, 0)))
    planes = bx.T.reshape(4, _R, _C)
    s = jnp.pad(scores, (0, pad), constant_values=-jnp.inf).reshape(_R, _C)
    thr = jnp.stack([jnp.asarray(iou_threshold, jnp.float32),
                     jnp.asarray(scores_threshold, jnp.float32)]).reshape(2, 1)

    sel_m, num_m = pl.pallas_call(
        _nms_kernel,
        in_specs=[
            pl.BlockSpec(memory_space=pltpu.SMEM),
            pl.BlockSpec(memory_space=pltpu.VMEM),
            pl.BlockSpec(memory_space=pltpu.VMEM),
            pl.BlockSpec(memory_space=pltpu.VMEM),
            pl.BlockSpec(memory_space=pltpu.VMEM),
            pl.BlockSpec(memory_space=pltpu.VMEM),
        ],
        out_specs=[
            pl.BlockSpec(memory_space=pltpu.VMEM),
            pl.BlockSpec(memory_space=pltpu.SMEM),
        ],
        out_shape=[
            jax.ShapeDtypeStruct((8, _C), jnp.int32),
            jax.ShapeDtypeStruct((1, 1), jnp.int32),
        ],
        scratch_shapes=[
            pltpu.VMEM((_R, _C), jnp.float32),
            pltpu.VMEM((_R, _C), jnp.float32),
        ],
    )(thr, planes[0], planes[1], planes[2], planes[3], s)

    sel = sel_m.reshape(-1)[:_MOUT]
    num = jnp.minimum(num_m[0, 0], jnp.asarray(max_output_size, jnp.int32))
    return (sel, num)


# speedup vs baseline: 1.6089x; 1.1988x over previous
import jax
import jax.numpy as jnp
from jax.experimental import pallas as pl
from jax.experimental.pallas import tpu as pltpu

_R, _C = 160, 128
_P = _R * _C
_G = _R // 8
_MOUT = 200
_BIG = 2**30
_B = 8


def _tile_iota():
    return (jax.lax.broadcasted_iota(jnp.int32, (8, _C), 0) * _C
            + jax.lax.broadcasted_iota(jnp.int32, (8, _C), 1))


def _combine(a, b):
    take_b = (b[0] > a[0]) | ((b[0] == a[0]) & (b[1] < a[1]))
    return (jnp.where(take_b, b[0], a[0]), jnp.where(take_b, b[1], a[1]))


def _nms_kernel(thr_ref, x1s, y1s, x2s, y2s, s, sel_ref, num_ref, ws):
    iou_thr = thr_ref[0, 0]
    score_thr = thr_ref[1, 0]
    ws[...] = jnp.where(s[...] > score_thr, s[...], -jnp.inf)

    ti = _tile_iota()

    def tree_sweep(idx, valid):
        acc0 = acc1 = None
        for g in range(_G):
            gs = pl.ds(g * 8, 8)
            w_g = ws[gs, :]
            if idx is not None:
                pick_g = ((ti + g * 1024) == idx) & valid
                w_g = jnp.where(pick_g, -jnp.inf, w_g)
                ws[gs, :] = w_g
            node = (w_g, ti + g * 1024)
            if g % 2 == 0:
                acc0 = node if acc0 is None else _combine(acc0, node)
            else:
                acc1 = node if acc1 is None else _combine(acc1, node)
        return _combine(acc0, acc1)

    m_v0, i_v0 = tree_sweep(None, None)

    def round_fn(state):
        num, stop, sel, sx1, sy1, sx2, sy2, sa, m_v, i_v = state
        m = jnp.max(m_v)
        valid = m > -jnp.inf
        idx = jnp.min(jnp.where(m_v == m, i_v, _BIG))
        r = idx // _C
        c = idx - r * _C
        zero = jnp.float32(0.0)
        b0 = x1s[r, c]
        b1 = y1s[r, c]
        b2 = x2s[r, c]
        b3 = y2s[r, c]
        a = (b2 - b0) * (b3 - b1)
        xx1 = jnp.maximum(b0, sx1)
        yy1 = jnp.maximum(b1, sy1)
        xx2 = jnp.minimum(b2, sx2)
        yy2 = jnp.minimum(b3, sy2)
        inter = (jnp.clip(xx2 - xx1, 0.0, None)
                 * jnp.clip(yy2 - yy1, 0.0, None))
        union = jnp.maximum(a + sa - inter, 1e-6)
        iou = inter / union
        hit = (iou >= iou_thr) & (ti < num)
        hitcnt = jnp.sum(jnp.where(hit, 1.0, zero))
        accepted = valid & (hitcnt == zero) & (num < _MOUT)
        slot = accepted & (ti == num)
        sel = jnp.where(slot, idx, sel)
        sx1 = jnp.where(slot, b0, sx1)
        sy1 = jnp.where(slot, b1, sy1)
        sx2 = jnp.where(slot, b2, sx2)
        sy2 = jnp.where(slot, b3, sy2)
        sa = jnp.where(slot, a, sa)
        num = num + accepted.astype(jnp.int32)
        m_v, i_v = tree_sweep(idx, valid)
        stop = jnp.logical_not(valid)
        return (num, stop, sel, sx1, sy1, sx2, sy2, sa, m_v, i_v)

    def cond(carry):
        return jnp.logical_and(carry[0] < _MOUT, jnp.logical_not(carry[1]))

    def body(carry):
        state = carry
        for _ in range(_B):
            state = round_fn(state)
        return state

    zf = jnp.zeros((8, _C), jnp.float32)
    carry = (jnp.int32(0), jnp.bool_(False),
             jnp.zeros((8, _C), jnp.int32), zf, zf, zf, zf, zf, m_v0, i_v0)
    carry = jax.lax.while_loop(cond, body, carry)
    sel_ref[...] = carry[2]
    num_ref[0, 0] = carry[0]


def kernel(boxes, scores, max_output_size, iou_threshold, scores_threshold):
    boxes = boxes.astype(jnp.float32)
    scores = scores.astype(jnp.float32)
    n = boxes.shape[0]
    pad = _P - n
    bx = jnp.pad(boxes, ((0, pad), (0, 0)))
    planes = bx.T.reshape(4, _R, _C)
    s = jnp.pad(scores, (0, pad), constant_values=-jnp.inf).reshape(_R, _C)
    thr = jnp.stack([jnp.asarray(iou_threshold, jnp.float32),
                     jnp.asarray(scores_threshold, jnp.float32)]).reshape(2, 1)

    sel_m, num_m = pl.pallas_call(
        _nms_kernel,
        in_specs=[
            pl.BlockSpec(memory_space=pltpu.SMEM),
            pl.BlockSpec(memory_space=pltpu.SMEM),
            pl.BlockSpec(memory_space=pltpu.SMEM),
            pl.BlockSpec(memory_space=pltpu.SMEM),
            pl.BlockSpec(memory_space=pltpu.SMEM),
            pl.BlockSpec(memory_space=pltpu.VMEM),
        ],
        out_specs=[
            pl.BlockSpec(memory_space=pltpu.VMEM),
            pl.BlockSpec(memory_space=pltpu.SMEM),
        ],
        out_shape=[
            jax.ShapeDtypeStruct((8, _C), jnp.int32),
            jax.ShapeDtypeStruct((1, 1), jnp.int32),
        ],
        scratch_shapes=[
            pltpu.VMEM((_R, _C), jnp.float32),
        ],
    )(thr, planes[0], planes[1], planes[2], planes[3], s)

    sel = sel_m.reshape(-1)[:_MOUT]
    num = jnp.minimum(num_m[0, 0], jnp.asarray(max_output_size, jnp.int32))
    return (sel, num)


# speedup vs baseline: 1.6306x; 1.0135x over previous
import jax
import jax.numpy as jnp
from jax.experimental import pallas as pl
from jax.experimental.pallas import tpu as pltpu

_R, _C = 160, 128
_P = _R * _C
_G = _R // 8
_MOUT = 200
_BIG = 2**30
_B = 16


def _tile_iota():
    return (jax.lax.broadcasted_iota(jnp.int32, (8, _C), 0) * _C
            + jax.lax.broadcasted_iota(jnp.int32, (8, _C), 1))


def _combine(a, b):
    take_b = (b[0] > a[0]) | ((b[0] == a[0]) & (b[1] < a[1]))
    return (jnp.where(take_b, b[0], a[0]), jnp.where(take_b, b[1], a[1]))


def _nms_kernel(thr_ref, x1s, y1s, x2s, y2s, s, sel_ref, num_ref, ws):
    iou_thr = thr_ref[0, 0]
    score_thr = thr_ref[1, 0]
    ws[...] = jnp.where(s[...] > score_thr, s[...], -jnp.inf)

    ti = _tile_iota()

    def tree_sweep(idx, valid):
        acc0 = acc1 = None
        for g in range(_G):
            gs = pl.ds(g * 8, 8)
            w_g = ws[gs, :]
            if idx is not None:
                pick_g = ((ti + g * 1024) == idx) & valid
                w_g = jnp.where(pick_g, -jnp.inf, w_g)
                ws[gs, :] = w_g
            node = (w_g, ti + g * 1024)
            if g % 2 == 0:
                acc0 = node if acc0 is None else _combine(acc0, node)
            else:
                acc1 = node if acc1 is None else _combine(acc1, node)
        return _combine(acc0, acc1)

    m_v0, i_v0 = tree_sweep(None, None)

    def round_fn(state):
        num, stop, sel, sx1, sy1, sx2, sy2, sa, m_v, i_v = state
        m = jnp.max(m_v)
        valid = m > -jnp.inf
        idx = jnp.min(jnp.where(m_v == m, i_v, _BIG))
        r = idx // _C
        c = idx - r * _C
        zero = jnp.float32(0.0)
        b0 = x1s[r, c]
        b1 = y1s[r, c]
        b2 = x2s[r, c]
        b3 = y2s[r, c]
        a = (b2 - b0) * (b3 - b1)
        nm_v, ni_v = tree_sweep(idx, valid)
        xx1 = jnp.maximum(b0, sx1)
        yy1 = jnp.maximum(b1, sy1)
        xx2 = jnp.minimum(b2, sx2)
        yy2 = jnp.minimum(b3, sy2)
        inter = (jnp.clip(xx2 - xx1, 0.0, None)
                 * jnp.clip(yy2 - yy1, 0.0, None))
        union = jnp.maximum(a + sa - inter, 1e-6)
        iou = inter / union
        hit = (iou >= iou_thr) & (ti < num)
        hitcnt = jnp.sum(jnp.where(hit, 1.0, zero))
        accepted = valid & (hitcnt == zero) & (num < _MOUT)
        slot = accepted & (ti == num)
        sel = jnp.where(slot, idx, sel)
        sx1 = jnp.where(slot, b0, sx1)
        sy1 = jnp.where(slot, b1, sy1)
        sx2 = jnp.where(slot, b2, sx2)
        sy2 = jnp.where(slot, b3, sy2)
        sa = jnp.where(slot, a, sa)
        num = num + accepted.astype(jnp.int32)
        stop = jnp.logical_not(valid)
        return (num, stop, sel, sx1, sy1, sx2, sy2, sa, nm_v, ni_v)

    def cond(carry):
        return jnp.logical_and(carry[0] < _MOUT, jnp.logical_not(carry[1]))

    def body(carry):
        state = carry
        for _ in range(_B):
            state = round_fn(state)
        return state

    zf = jnp.zeros((8, _C), jnp.float32)
    carry = (jnp.int32(0), jnp.bool_(False),
             jnp.zeros((8, _C), jnp.int32), zf, zf, zf, zf, zf, m_v0, i_v0)
    carry = jax.lax.while_loop(cond, body, carry)
    sel_ref[...] = carry[2]
    num_ref[0, 0] = carry[0]


def kernel(boxes, scores, max_output_size, iou_threshold, scores_threshold):
    boxes = boxes.astype(jnp.float32)
    scores = scores.astype(jnp.float32)
    n = boxes.shape[0]
    pad = _P - n
    bx = jnp.pad(boxes, ((0, pad), (0, 0)))
    planes = bx.T.reshape(4, _R, _C)
    s = jnp.pad(scores, (0, pad), constant_values=-jnp.inf).reshape(_R, _C)
    thr = jnp.stack([jnp.asarray(iou_threshold, jnp.float32),
                     jnp.asarray(scores_threshold, jnp.float32)]).reshape(2, 1)

    sel_m, num_m = pl.pallas_call(
        _nms_kernel,
        in_specs=[
            pl.BlockSpec(memory_space=pltpu.SMEM),
            pl.BlockSpec(memory_space=pltpu.SMEM),
            pl.BlockSpec(memory_space=pltpu.SMEM),
            pl.BlockSpec(memory_space=pltpu.SMEM),
            pl.BlockSpec(memory_space=pltpu.SMEM),
            pl.BlockSpec(memory_space=pltpu.VMEM),
        ],
        out_specs=[
            pl.BlockSpec(memory_space=pltpu.VMEM),
            pl.BlockSpec(memory_space=pltpu.SMEM),
        ],
        out_shape=[
            jax.ShapeDtypeStruct((8, _C), jnp.int32),
            jax.ShapeDtypeStruct((1, 1), jnp.int32),
        ],
        scratch_shapes=[
            pltpu.VMEM((_R, _C), jnp.float32),
        ],
    )(thr, planes[0], planes[1], planes[2], planes[3], s)

    sel = sel_m.reshape(-1)[:_MOUT]
    num = jnp.minimum(num_m[0, 0], jnp.asarray(max_output_size, jnp.int32))
    return (sel, num)
